# Initial kernel scaffold; baseline (speedup 1.0000x reference)
import jax
import jax.numpy as jnp
from jax.experimental import pallas as pl


def kernel(x_user, x_item, edge_index_u2i, edge_index_i2u, Wsrc_0_u2i, Wdst_0_u2i, asrc_0_u2i, adst_0_u2i, b_0_u2i, Wsrc_0_i2u, Wdst_0_i2u, asrc_0_i2u, adst_0_i2u, b_0_i2u, lnw_0_user, lnb_0_user, lnw_0_item, lnb_0_item, Wsrc_1_u2i, Wdst_1_u2i, asrc_1_u2i, adst_1_u2i, b_1_u2i, Wsrc_1_i2u, Wdst_1_i2u, asrc_1_i2u, adst_1_i2u, b_1_i2u, lnw_1_user, lnb_1_user, lnw_1_item, lnb_1_item):
    raise NotImplementedError("write your pallas kernel here")


# speedup vs baseline: 5.3334x; 5.3334x over previous
import functools

import jax
import jax.numpy as jnp
from jax import lax
from jax.experimental import pallas as pl
from jax.experimental.pallas import tpu as pltpu
from jax.experimental.pallas import tpu_sc as plsc

N = 10000
C = 128
H = 8
HC = H * C
E = 160000
AW = 16
GW = 128
NC = 2
NS = 16
NW = NC * NS
EPW = E // NW
KA = 40
KB = 8
NP = 10240
RPT = NP // NS

_mesh = plsc.VectorSubcoreMesh(core_axis_name="c", subcore_axis_name="s")


def _proj_body(xs_ref, xd_ref, ws_ref, wd_ref, asf_ref, adf_ref,
               hs_ref, as_ref, ad_ref):
    xs = xs_ref[...]
    ws = ws_ref[...]
    hs_ref[...] = jnp.dot(xs, ws, preferred_element_type=jnp.float32)
    rows = lax.broadcasted_iota(jnp.int32, (HC, GW), 0)
    cols = lax.broadcasted_iota(jnp.int32, (HC, GW), 1)
    sel = jnp.where(rows // C == cols, 1.0, 0.0).astype(jnp.float32)
    wsf = jnp.dot(ws * asf_ref[...], sel, preferred_element_type=jnp.float32)
    as_ref[...] = jnp.dot(xs, wsf, preferred_element_type=jnp.float32)
    wdf = jnp.dot(wd_ref[...] * adf_ref[...], sel,
                  preferred_element_type=jnp.float32)
    ad_ref[...] = jnp.dot(xd_ref[...], wdf, preferred_element_type=jnp.float32)


def _project(x_src, x_dst, Wsrc, Wdst, asrc, adst):
    nb = 1000
    grid = (N // nb,)
    return pl.pallas_call(
        _proj_body,
        grid=grid,
        in_specs=[
            pl.BlockSpec((nb, C), lambda i: (i, 0)),
            pl.BlockSpec((nb, C), lambda i: (i, 0)),
            pl.BlockSpec((C, HC), lambda i: (0, 0)),
            pl.BlockSpec((C, HC), lambda i: (0, 0)),
            pl.BlockSpec((1, HC), lambda i: (0, 0)),
            pl.BlockSpec((1, HC), lambda i: (0, 0)),
        ],
        out_specs=[
            pl.BlockSpec((nb, HC), lambda i: (i, 0)),
            pl.BlockSpec((nb, GW), lambda i: (i, 0)),
            pl.BlockSpec((nb, GW), lambda i: (i, 0)),
        ],
        out_shape=[
            jax.ShapeDtypeStruct((N, HC), jnp.float32),
            jax.ShapeDtypeStruct((N, GW), jnp.float32),
            jax.ShapeDtypeStruct((N, GW), jnp.float32),
        ],
    )(x_src, x_dst, Wsrc, Wdst, asrc.reshape(1, HC), adst.reshape(1, HC))


@functools.partial(
    pl.kernel,
    mesh=_mesh,
    out_type=(
        jax.ShapeDtypeStruct((E, AW), jnp.float32),
        jax.ShapeDtypeStruct((NC, NP, GW), jnp.float32),
    ),
    scratch_types=[
        pltpu.VMEM((KA,), jnp.int32),
        pltpu.VMEM((KA,), jnp.int32),
        pltpu.VMEM((KA, GW), jnp.float32),
        pltpu.VMEM((KA, GW), jnp.float32),
        pltpu.VMEM((KA, AW), jnp.float32),
        pltpu.VMEM((KA, GW), jnp.float32),
        pltpu.VMEM_SHARED((NP, GW), jnp.float32),
        pltpu.SemaphoreType.DMA,
        pltpu.SemaphoreType.DMA,
    ],
)
def _phase_a(src_hbm, dst_hbm, as_hbm, ad_hbm, ex_hbm, spart_hbm,
             idx_s, idx_d, asr, adr, exb, exw, s_acc, sem1, sem2):
    cid = lax.axis_index("c")
    sid = lax.axis_index("s")
    wid = sid * NC + cid

    def zbody(k, _):
        for j in range(GW // AW):
            exw[k, pl.ds(j * AW, AW)] = jnp.zeros((AW,), jnp.float32)
        return 0
    lax.fori_loop(0, KA, zbody, 0)

    def zcopy(t, _):
        pltpu.sync_copy(exw, s_acc.at[pl.ds(sid * RPT + t * KA, KA)])
        return 0
    lax.fori_loop(0, RPT // KA, zcopy, 0)
    plsc.subcore_barrier()

    def body(i, _):
        base = wid * EPW + i * KA
        pltpu.sync_copy(src_hbm.at[pl.ds(base, KA)], idx_s)
        pltpu.sync_copy(dst_hbm.at[pl.ds(base, KA)], idx_d)
        pltpu.async_copy(as_hbm.at[idx_s], asr, sem1).wait()
        pltpu.async_copy(ad_hbm.at[idx_d], adr, sem2).wait()

        def ebody(k, _):
            v = asr[k, pl.ds(0, AW)] + adr[k, pl.ds(0, AW)]
            v = jnp.maximum(v, 0.2 * v)
            v = jnp.exp(v)
            exb[k, :] = v
            exw[k, pl.ds(0, AW)] = v
            return 0
        lax.fori_loop(0, KA, ebody, 0)
        pltpu.sync_copy(exb, ex_hbm.at[pl.ds(base, KA)])
        pltpu.sync_copy(exw, s_acc.at[idx_d], add=True)
        return 0
    lax.fori_loop(0, EPW // KA, body, 0)

    plsc.subcore_barrier()

    def wcopy(t, _):
        r0 = sid * RPT + t * KA
        pltpu.sync_copy(s_acc.at[pl.ds(r0, KA)], asr)
        pltpu.sync_copy(asr, spart_hbm.at[cid, pl.ds(r0, KA)])
        return 0
    lax.fori_loop(0, RPT // KA, wcopy, 0)


@functools.partial(
    pl.kernel,
    mesh=_mesh,
    out_type=jax.ShapeDtypeStruct((NC, NP, C), jnp.float32),
    scratch_types=[
        pltpu.VMEM((KB,), jnp.int32),
        pltpu.VMEM((KB,), jnp.int32),
        pltpu.VMEM((KB, AW), jnp.float32),
        pltpu.VMEM((KB, GW), jnp.float32),
        pltpu.VMEM((KB, GW), jnp.float32),
        pltpu.VMEM((KB, HC), jnp.float32),
        pltpu.VMEM((KB, C), jnp.float32),
        pltpu.VMEM_SHARED((NP, C), jnp.float32),
        pltpu.SemaphoreType.DMA,
        pltpu.SemaphoreType.DMA,
        pltpu.SemaphoreType.DMA,
    ],
)
def _phase_b(src_hbm, dst_hbm, ex_hbm, s0_hbm, s1_hbm, hs_hbm, opart_hbm,
             idx_s, idx_d, exb, s0r, s1r, hsb, msgb, out_acc,
             sem1, sem2, sem3):
    cid = lax.axis_index("c")
    sid = lax.axis_index("s")
    wid = sid * NC + cid

    def zbody(k, _):
        for j in range(C // AW):
            msgb[k, pl.ds(j * AW, AW)] = jnp.zeros((AW,), jnp.float32)
        return 0
    lax.fori_loop(0, KB, zbody, 0)

    def zcopy(t, _):
        pltpu.sync_copy(msgb, out_acc.at[pl.ds(sid * RPT + t * KB, KB)])
        return 0
    lax.fori_loop(0, RPT // KB, zcopy, 0)
    plsc.subcore_barrier()

    def body(i, _):
        base = wid * EPW + i * KB
        pltpu.sync_copy(src_hbm.at[pl.ds(base, KB)], idx_s)
        pltpu.sync_copy(dst_hbm.at[pl.ds(base, KB)], idx_d)
        pltpu.sync_copy(ex_hbm.at[pl.ds(base, KB)], exb)
        pltpu.async_copy(hs_hbm.at[idx_s], hsb, sem1).wait()
        pltpu.async_copy(s0_hbm.at[idx_d], s0r, sem2).wait()
        pltpu.async_copy(s1_hbm.at[idx_d], s1r, sem3).wait()

        def ebody(k, _):
            sv = s0r[k, pl.ds(0, AW)] + s1r[k, pl.ds(0, AW)]
            av = exb[k, :] * (1.0 / H) / (sv + 1e-16)
            accs = [jnp.zeros((AW,), jnp.float32) for _ in range(C // AW)]
            for h in range(H):
                a = av[h]
                for j in range(C // AW):
                    accs[j] = accs[j] + a * hsb[k, pl.ds(h * C + j * AW, AW)]
            for j in range(C // AW):
                msgb[k, pl.ds(j * AW, AW)] = accs[j]
            return 0
        lax.fori_loop(0, KB, ebody, 0)
        pltpu.sync_copy(msgb, out_acc.at[idx_d], add=True)
        return 0
    lax.fori_loop(0, EPW // KB, body, 0)

    plsc.subcore_barrier()

    def wcopy(t, _):
        r0 = sid * RPT + t * KB
        pltpu.sync_copy(out_acc.at[pl.ds(r0, KB)], msgb)
        pltpu.sync_copy(msgb, opart_hbm.at[cid, pl.ds(r0, KB)])
        return 0
    lax.fori_loop(0, RPT // KB, wcopy, 0)


def _fin_body(p_ref, b_ref, w_ref, lb_ref, o_ref):
    t = p_ref[0] + p_ref[1] + b_ref[...]
    mu = jnp.mean(t, axis=-1, keepdims=True)
    var = jnp.mean((t - mu) * (t - mu), axis=-1, keepdims=True)
    y = (t - mu) / jnp.sqrt(var + 1e-5) * w_ref[...] + lb_ref[...]
    o_ref[...] = jnp.maximum(y, 0.0)


def _finalize(parts, b, lnw, lnb):
    nb = 1000
    return pl.pallas_call(
        _fin_body,
        grid=(N // nb,),
        in_specs=[
            pl.BlockSpec((NC, nb, C), lambda i: (0, i, 0)),
            pl.BlockSpec((1, C), lambda i: (0, 0)),
            pl.BlockSpec((1, C), lambda i: (0, 0)),
            pl.BlockSpec((1, C), lambda i: (0, 0)),
        ],
        out_specs=pl.BlockSpec((nb, C), lambda i: (i, 0)),
        out_shape=jax.ShapeDtypeStruct((N, C), jnp.float32),
    )(parts, b.reshape(1, C), lnw.reshape(1, C), lnb.reshape(1, C))


def _conv(x_src, x_dst, src, dst, Wsrc, Wdst, asrc, adst):
    hs, a_s, a_d = _project(x_src, x_dst, Wsrc, Wdst, asrc, adst)
    ex, spart = _phase_a(src, dst, a_s, a_d)
    opart = _phase_b(src, dst, ex, spart[0], spart[1], hs)
    return opart


def kernel(x_user, x_item, edge_index_u2i, edge_index_i2u,
           Wsrc_0_u2i, Wdst_0_u2i, asrc_0_u2i, adst_0_u2i, b_0_u2i,
           Wsrc_0_i2u, Wdst_0_i2u, asrc_0_i2u, adst_0_i2u, b_0_i2u,
           lnw_0_user, lnb_0_user, lnw_0_item, lnb_0_item,
           Wsrc_1_u2i, Wdst_1_u2i, asrc_1_u2i, adst_1_u2i, b_1_u2i,
           Wsrc_1_i2u, Wdst_1_i2u, asrc_1_i2u, adst_1_i2u, b_1_i2u,
           lnw_1_user, lnb_1_user, lnw_1_item, lnb_1_item):
    p = dict(locals())
    s_u2i, d_u2i = edge_index_u2i[0], edge_index_u2i[1]
    s_i2u, d_i2u = edge_index_i2u[0], edge_index_i2u[1]
    xu, xi = x_user, x_item
    for l in range(2):
        op_i = _conv(xu, xi, s_u2i, d_u2i,
                     p[f"Wsrc_{l}_u2i"], p[f"Wdst_{l}_u2i"],
                     p[f"asrc_{l}_u2i"], p[f"adst_{l}_u2i"])
        op_u = _conv(xi, xu, s_i2u, d_i2u,
                     p[f"Wsrc_{l}_i2u"], p[f"Wdst_{l}_i2u"],
                     p[f"asrc_{l}_i2u"], p[f"adst_{l}_i2u"])
        xi = _finalize(op_i, p[f"b_{l}_u2i"], p[f"lnw_{l}_item"],
                       p[f"lnb_{l}_item"])
        xu = _finalize(op_u, p[f"b_{l}_i2u"], p[f"lnw_{l}_user"],
                       p[f"lnb_{l}_user"])
    return jnp.stack([xu, xi], axis=0)


# speedup vs baseline: 15.1350x; 2.8378x over previous
import functools

import jax
import jax.numpy as jnp
from jax import lax
from jax.experimental import pallas as pl
from jax.experimental.pallas import tpu as pltpu
from jax.experimental.pallas import tpu_sc as plsc

N = 10000
C = 128
H = 8
HC = H * C
E = 160000
EP = 163840
AW = 16
GW = 128
NC = 2
NS = 16
NW = NC * NS
EPW = EP // NW
NP = 10240
RPT = NP // NS

KEA = 32
NITA = EPW // KEA
ICA = 8
KE2 = 128
NIT2 = EPW // KE2
KEB = 16
NITB = EPW // KEB
ICB = 16

_mesh = plsc.VectorSubcoreMesh(core_axis_name="c", subcore_axis_name="s")


def _proj_body(xs_ref, xd_ref, ws_ref, wd_ref, asf_ref, adf_ref,
               hs_ref, as_ref, ad_ref):
    xs = xs_ref[...]
    ws = ws_ref[...]
    hs = jnp.dot(xs, ws, preferred_element_type=jnp.float32)

    def rne16(x):
        xi = lax.bitcast_convert_type(x, jnp.int32)
        return ((xi + 0x7FFF + ((xi >> 16) & 1)) >> 16) & 0xFFFF

    packs = []
    for h in range(H):
        a = hs[:, h * C:h * C + C // 2]
        bb = hs[:, h * C + C // 2:(h + 1) * C]
        packs.append((rne16(bb) << 16) | rne16(a))
    hs_ref[...] = jnp.concatenate(packs, axis=1)
    rows = lax.broadcasted_iota(jnp.int32, (HC, GW), 0)
    cols = lax.broadcasted_iota(jnp.int32, (HC, GW), 1)
    sel = jnp.where(rows // C == cols, 1.0, 0.0).astype(jnp.float32)
    wsf = jnp.dot(ws * asf_ref[...], sel, preferred_element_type=jnp.float32)
    as_ref[...] = jnp.dot(xs, wsf, preferred_element_type=jnp.float32)
    wdf = jnp.dot(wd_ref[...] * adf_ref[...], sel,
                  preferred_element_type=jnp.float32)
    ad_ref[...] = jnp.dot(xd_ref[...], wdf, preferred_element_type=jnp.float32)


def _project(x_src, x_dst, Wsrc, Wdst, asrc, adst):
    nb = 1000
    return pl.pallas_call(
        _proj_body,
        grid=(N // nb,),
        in_specs=[
            pl.BlockSpec((nb, C), lambda i: (i, 0)),
            pl.BlockSpec((nb, C), lambda i: (i, 0)),
            pl.BlockSpec((C, HC), lambda i: (0, 0)),
            pl.BlockSpec((C, HC), lambda i: (0, 0)),
            pl.BlockSpec((1, HC), lambda i: (0, 0)),
            pl.BlockSpec((1, HC), lambda i: (0, 0)),
        ],
        out_specs=[
            pl.BlockSpec((nb, HC // 2), lambda i: (i, 0)),
            pl.BlockSpec((nb, GW), lambda i: (i, 0)),
            pl.BlockSpec((nb, GW), lambda i: (i, 0)),
        ],
        out_shape=[
            jax.ShapeDtypeStruct((N, HC // 2), jnp.int32),
            jax.ShapeDtypeStruct((N, GW), jnp.float32),
            jax.ShapeDtypeStruct((N, GW), jnp.float32),
        ],
    )(x_src, x_dst, Wsrc, Wdst, asrc.reshape(1, HC), adst.reshape(1, HC))


@functools.partial(
    pl.kernel,
    mesh=_mesh,
    out_type=(
        jax.ShapeDtypeStruct((EP, AW), jnp.float32),
        jax.ShapeDtypeStruct((NC, NP, GW), jnp.float32),
    ),
    scratch_types=[
        pltpu.VMEM((ICA, KEA), jnp.int32),
        pltpu.VMEM((ICA, KEA), jnp.int32),
        pltpu.VMEM((2, KEA, GW), jnp.float32),
        pltpu.VMEM((2, KEA, GW), jnp.float32),
        pltpu.VMEM((KEA, AW), jnp.float32),
        pltpu.VMEM((KEA, GW), jnp.float32),
        pltpu.VMEM_SHARED((NP, GW), jnp.float32),
        pltpu.SemaphoreType.DMA((2,)),
        pltpu.SemaphoreType.DMA((2,)),
    ],
)
def _phase_a(src_hbm, dst_hbm, as_hbm, ad_hbm, ex_hbm, spart_hbm,
             idx_s, idx_d, asr, adr, exb, exw, s_acc, sem1, sem2):
    cid = lax.axis_index("c")
    sid = lax.axis_index("s")
    wid = sid * NC + cid

    def zbody(k, _):
        for j in range(GW // AW):
            exw[k, pl.ds(j * AW, AW)] = jnp.zeros((AW,), jnp.float32)
        return 0
    lax.fori_loop(0, KEA, zbody, 0)

    def zcopy(t, _):
        pltpu.sync_copy(exw, s_acc.at[pl.ds(sid * RPT + t * KEA, KEA)])
        return 0
    lax.fori_loop(0, RPT // KEA, zcopy, 0)
    plsc.subcore_barrier()

    def issue(ii, b):
        ca = pltpu.async_copy(as_hbm.at[idx_s.at[ii]], asr.at[b], sem1.at[b])
        cb = pltpu.async_copy(ad_hbm.at[idx_d.at[ii]], adr.at[b], sem2.at[b])
        return ca, cb

    def wait(ii, b):
        pltpu.make_async_copy(as_hbm.at[idx_s.at[ii]], asr.at[b],
                              sem1.at[b]).wait()
        pltpu.make_async_copy(ad_hbm.at[idx_d.at[ii]], adr.at[b],
                              sem2.at[b]).wait()

    def outer(c, _):
        pltpu.sync_copy(src_hbm.at[wid, pl.ds(c * ICA, ICA)], idx_s)
        pltpu.sync_copy(dst_hbm.at[wid, pl.ds(c * ICA, ICA)], idx_d)
        issue(0, 0)

        def inner(j, _):
            for b in range(2):
                ii = 2 * j + b
                i = c * ICA + ii
                wait(ii, b)

                @pl.when(ii < ICA - 1)
                def _():
                    issue(ii + 1, 1 - b)

                def ebody(k, _):
                    v = (asr.at[b][k, pl.ds(0, AW)] + adr.at[b][k, pl.ds(0, AW)])
                    v = jnp.maximum(v, 0.2 * v)
                    v = jnp.exp(v)
                    exb[k, :] = v
                    exw[k, pl.ds(0, AW)] = v
                    return 0
                lax.fori_loop(0, KEA, ebody, 0)
                base = wid * EPW + i * KEA
                pltpu.sync_copy(exb, ex_hbm.at[pl.ds(base, KEA)])
                pltpu.sync_copy(exw, s_acc.at[idx_d.at[ii]], add=True)
            return 0
        lax.fori_loop(0, ICA // 2, inner, 0)
        return 0
    lax.fori_loop(0, NITA // ICA, outer, 0)

    plsc.subcore_barrier()

    def wcopy(t, _):
        r0 = sid * RPT + t * KEA
        pltpu.sync_copy(s_acc.at[pl.ds(r0, KEA)], asr.at[0])
        pltpu.sync_copy(asr.at[0], spart_hbm.at[cid, pl.ds(r0, KEA)])
        return 0
    lax.fori_loop(0, RPT // KEA, wcopy, 0)


def _ssum_body(p_ref, o_ref):
    o_ref[...] = p_ref[0] + p_ref[1]


def _ssum(parts):
    nb = 1024
    return pl.pallas_call(
        _ssum_body,
        grid=(NP // nb,),
        in_specs=[pl.BlockSpec((NC, nb, GW), lambda i: (0, i, 0))],
        out_specs=pl.BlockSpec((nb, GW), lambda i: (i, 0)),
        out_shape=jax.ShapeDtypeStruct((NP, GW), jnp.float32),
    )(parts)


@functools.partial(
    pl.kernel,
    mesh=_mesh,
    out_type=jax.ShapeDtypeStruct((EP, AW), jnp.float32),
    scratch_types=[
        pltpu.VMEM((NIT2, KE2), jnp.int32),
        pltpu.VMEM((2, KE2, GW), jnp.float32),
        pltpu.VMEM((KE2, AW), jnp.float32),
        pltpu.VMEM((KE2, AW), jnp.float32),
        pltpu.SemaphoreType.DMA((2,)),
    ],
)
def _phase_a2(dst_hbm, ex_hbm, s_hbm, al_hbm,
              idx_d, ssr, exb, alb, sem):
    cid = lax.axis_index("c")
    sid = lax.axis_index("s")
    wid = sid * NC + cid

    pltpu.sync_copy(dst_hbm.at[wid], idx_d)
    pltpu.async_copy(s_hbm.at[idx_d.at[0]], ssr.at[0], sem.at[0])

    def body(j, _):
        for b in range(2):
            i = 2 * j + b

            @pl.when(i < NIT2 - 1)
            def _():
                pltpu.async_copy(s_hbm.at[idx_d.at[i + 1]], ssr.at[1 - b],
                                 sem.at[1 - b])
            pltpu.make_async_copy(s_hbm.at[idx_d.at[i]], ssr.at[b],
                                  sem.at[b]).wait()
            base = wid * EPW + i * KE2
            pltpu.sync_copy(ex_hbm.at[pl.ds(base, KE2)], exb)

            def ebody(k, _):
                sv = ssr.at[b][k, pl.ds(0, AW)]
                alb[k, :] = exb[k, :] * (1.0 / H) / (sv + 1e-16)
                return 0
            lax.fori_loop(0, KE2, ebody, 0)
            pltpu.sync_copy(alb, al_hbm.at[pl.ds(base, KE2)])
        return 0
    lax.fori_loop(0, NIT2 // 2, body, 0)


@functools.partial(
    pl.kernel,
    mesh=_mesh,
    out_type=jax.ShapeDtypeStruct((NC, NP, C), jnp.float32),
    scratch_types=[
        pltpu.VMEM((ICB, KEB), jnp.int32),
        pltpu.VMEM((ICB, KEB), jnp.int32),
        pltpu.VMEM((2, KEB, AW), jnp.float32),
        pltpu.VMEM((2, KEB, HC // 2), jnp.int32),
        pltpu.VMEM((KEB, C), jnp.float32),
        pltpu.VMEM_SHARED((NP, C), jnp.float32),
        pltpu.SemaphoreType.DMA((2,)),
        pltpu.SemaphoreType.DMA((2,)),
    ],
)
def _phase_b(src_hbm, dst_hbm, al_hbm, hs_hbm, opart_hbm,
             idx_s, idx_d, alb, hsb, msgb, out_acc, sem1, sem2):
    cid = lax.axis_index("c")
    sid = lax.axis_index("s")
    wid = sid * NC + cid

    def zbody(k, _):
        for j in range(C // AW):
            msgb[k, pl.ds(j * AW, AW)] = jnp.zeros((AW,), jnp.float32)
        return 0
    lax.fori_loop(0, KEB, zbody, 0)

    def zcopy(t, _):
        pltpu.sync_copy(msgb, out_acc.at[pl.ds(sid * RPT + t * KEB, KEB)])
        return 0
    lax.fori_loop(0, RPT // KEB, zcopy, 0)
    plsc.subcore_barrier()

    def issue(i, ii, b):
        base = wid * EPW + i * KEB
        ca = pltpu.async_copy(hs_hbm.at[idx_s.at[ii]], hsb.at[b], sem1.at[b])
        cb = pltpu.async_copy(al_hbm.at[pl.ds(base, KEB)], alb.at[b],
                              sem2.at[b])
        return ca, cb

    def wait(i, ii, b):
        base = wid * EPW + i * KEB
        pltpu.make_async_copy(hs_hbm.at[idx_s.at[ii]], hsb.at[b],
                              sem1.at[b]).wait()
        pltpu.make_async_copy(al_hbm.at[pl.ds(base, KEB)], alb.at[b],
                              sem2.at[b]).wait()

    def outer(c, _):
        pltpu.sync_copy(src_hbm.at[wid, pl.ds(c * ICB, ICB)], idx_s)
        pltpu.sync_copy(dst_hbm.at[wid, pl.ds(c * ICB, ICB)], idx_d)
        issue(c * ICB, 0, 0)

        def inner(j, _):
            for b in range(2):
                ii = 2 * j + b
                i = c * ICB + ii
                wait(i, ii, b)

                @pl.when(ii < ICB - 1)
                def _():
                    issue(i + 1, ii + 1, 1 - b)

                for k in range(KEB):
                    av = alb.at[b][k, :]
                    acca = [jnp.zeros((AW,), jnp.float32) for _ in range(4)]
                    accb = [jnp.zeros((AW,), jnp.float32) for _ in range(4)]
                    for h in range(H):
                        a = av[h]
                        for m in range(4):
                            vi = hsb.at[b][k, pl.ds(h * 64 + m * AW, AW)]
                            ua = lax.bitcast_convert_type(
                                vi << 16, jnp.float32)
                            ub = lax.bitcast_convert_type(
                                vi & jnp.int32(-65536), jnp.float32)
                            acca[m] = acca[m] + a * ua
                            accb[m] = accb[m] + a * ub
                    for m in range(4):
                        msgb[k, pl.ds(m * AW, AW)] = acca[m]
                        msgb[k, pl.ds(64 + m * AW, AW)] = accb[m]
                pltpu.sync_copy(msgb, out_acc.at[idx_d.at[ii]], add=True)
            return 0
        lax.fori_loop(0, ICB // 2, inner, 0)
        return 0
    lax.fori_loop(0, NITB // ICB, outer, 0)

    plsc.subcore_barrier()

    def wcopy(t, _):
        r0 = sid * RPT + t * KEB
        pltpu.sync_copy(out_acc.at[pl.ds(r0, KEB)], msgb)
        pltpu.sync_copy(msgb, opart_hbm.at[cid, pl.ds(r0, KEB)])
        return 0
    lax.fori_loop(0, RPT // KEB, wcopy, 0)


def _fin_body(p_ref, b_ref, w_ref, lb_ref, o_ref):
    t = p_ref[0] + p_ref[1] + b_ref[...]
    mu = jnp.mean(t, axis=-1, keepdims=True)
    var = jnp.mean((t - mu) * (t - mu), axis=-1, keepdims=True)
    y = (t - mu) / jnp.sqrt(var + 1e-5) * w_ref[...] + lb_ref[...]
    o_ref[...] = jnp.maximum(y, 0.0)


def _finalize(parts, b, lnw, lnb):
    nb = 1000
    return pl.pallas_call(
        _fin_body,
        grid=(N // nb,),
        in_specs=[
            pl.BlockSpec((NC, nb, C), lambda i: (0, i, 0)),
            pl.BlockSpec((1, C), lambda i: (0, 0)),
            pl.BlockSpec((1, C), lambda i: (0, 0)),
            pl.BlockSpec((1, C), lambda i: (0, 0)),
        ],
        out_specs=pl.BlockSpec((nb, C), lambda i: (i, 0)),
        out_shape=jax.ShapeDtypeStruct((N, C), jnp.float32),
    )(parts, b.reshape(1, C), lnw.reshape(1, C), lnb.reshape(1, C))


def _conv(x_src, x_dst, ei, Wsrc, Wdst, asrc, adst):
    sA, dA, d2, sB, dB = ei
    hs, a_s, a_d = _project(x_src, x_dst, Wsrc, Wdst, asrc, adst)
    ex, spart = _phase_a(sA, dA, a_s, a_d)
    s_sum = _ssum(spart)
    alpha = _phase_a2(d2, ex, s_sum)
    opart = _phase_b(sB, dB, alpha, hs)
    return opart


def _pad_edges(ei):
    npad = EP - E
    src = jnp.concatenate(
        [ei[0], (jnp.arange(npad, dtype=jnp.int32) * 37) % N])
    dst = jnp.concatenate(
        [ei[1], N + (jnp.arange(npad, dtype=jnp.int32) % (NP - N))])
    return (src.reshape(NW, NITA, KEA), dst.reshape(NW, NITA, KEA),
            dst.reshape(NW, NIT2, KE2),
            src.reshape(NW, NITB, KEB), dst.reshape(NW, NITB, KEB))


def kernel(x_user, x_item, edge_index_u2i, edge_index_i2u,
           Wsrc_0_u2i, Wdst_0_u2i, asrc_0_u2i, adst_0_u2i, b_0_u2i,
           Wsrc_0_i2u, Wdst_0_i2u, asrc_0_i2u, adst_0_i2u, b_0_i2u,
           lnw_0_user, lnb_0_user, lnw_0_item, lnb_0_item,
           Wsrc_1_u2i, Wdst_1_u2i, asrc_1_u2i, adst_1_u2i, b_1_u2i,
           Wsrc_1_i2u, Wdst_1_i2u, asrc_1_i2u, adst_1_i2u, b_1_i2u,
           lnw_1_user, lnb_1_user, lnw_1_item, lnb_1_item):
    p = dict(locals())
    ei_u2i = _pad_edges(edge_index_u2i)
    ei_i2u = _pad_edges(edge_index_i2u)
    xu, xi = x_user, x_item
    for l in range(2):
        op_i = _conv(xu, xi, ei_u2i,
                     p[f"Wsrc_{l}_u2i"], p[f"Wdst_{l}_u2i"],
                     p[f"asrc_{l}_u2i"], p[f"adst_{l}_u2i"])
        op_u = _conv(xi, xu, ei_i2u,
                     p[f"Wsrc_{l}_i2u"], p[f"Wdst_{l}_i2u"],
                     p[f"asrc_{l}_i2u"], p[f"adst_{l}_i2u"])
        xi = _finalize(op_i, p[f"b_{l}_u2i"], p[f"lnw_{l}_item"],
                       p[f"lnb_{l}_item"])
        xu = _finalize(op_u, p[f"b_{l}_i2u"], p[f"lnw_{l}_user"],
                       p[f"lnb_{l}_user"])
    return jnp.stack([xu, xi], axis=0)


# speedup vs baseline: 15.8597x; 1.0479x over previous
import functools

import jax
import jax.numpy as jnp
from jax import lax
from jax.experimental import pallas as pl
from jax.experimental.pallas import tpu as pltpu
from jax.experimental.pallas import tpu_sc as plsc

N = 10000
C = 128
H = 8
HC = H * C
HP = HC // 2
E = 160000
EP = 163840
EPR = EP // 8
AW = 16
GW = 128
NC = 2
NS = 16
NW = NC * NS
EPW = EP // NW
EPWR = EPW // 8
NP = 10240
RPT = NP // NS

KEA = 32
NITA = EPW // KEA
ICA = 8
KE2 = 128
NIT2 = EPW // KE2
KEB = 16
NITB = EPW // KEB
ICB = 16

_mesh = plsc.VectorSubcoreMesh(core_axis_name="c", subcore_axis_name="s")


def _proj_body(xs_ref, xd_ref, ws_ref, wd_ref, asf_ref, adf_ref,
               hs_ref, as_ref, ad_ref):
    xs = xs_ref[...]
    ws = ws_ref[...]
    hs = jnp.dot(xs, ws, preferred_element_type=jnp.float32)

    def rne16(x):
        xi = lax.bitcast_convert_type(x, jnp.int32)
        return ((xi + 0x7FFF + ((xi >> 16) & 1)) >> 16) & 0xFFFF

    packs = []
    for h in range(H):
        a = hs[:, h * C:h * C + C // 2]
        bb = hs[:, h * C + C // 2:(h + 1) * C]
        packs.append((rne16(bb) << 16) | rne16(a))
    hs_ref[...] = jnp.concatenate(packs, axis=1)

    rows = lax.broadcasted_iota(jnp.int32, (HC, GW), 0)
    cols = lax.broadcasted_iota(jnp.int32, (HC, GW), 1)
    sel = jnp.where(rows // C == cols, 1.0, 0.0).astype(jnp.float32)
    wsf = jnp.dot(ws * asf_ref[...], sel, preferred_element_type=jnp.float32)
    as_ref[...] = jnp.dot(xs, wsf, preferred_element_type=jnp.float32)
    wdf = jnp.dot(wd_ref[...] * adf_ref[...], sel,
                  preferred_element_type=jnp.float32)
    ad_ref[...] = jnp.dot(xd_ref[...], wdf, preferred_element_type=jnp.float32)


def _project(x_src, x_dst, Wsrc, Wdst, asrc, adst):
    nb = 1000
    return pl.pallas_call(
        _proj_body,
        grid=(N // nb,),
        in_specs=[
            pl.BlockSpec((nb, C), lambda i: (i, 0)),
            pl.BlockSpec((nb, C), lambda i: (i, 0)),
            pl.BlockSpec((C, HC), lambda i: (0, 0)),
            pl.BlockSpec((C, HC), lambda i: (0, 0)),
            pl.BlockSpec((1, HC), lambda i: (0, 0)),
            pl.BlockSpec((1, HC), lambda i: (0, 0)),
        ],
        out_specs=[
            pl.BlockSpec((nb, HP), lambda i: (i, 0)),
            pl.BlockSpec((nb, GW), lambda i: (i, 0)),
            pl.BlockSpec((nb, GW), lambda i: (i, 0)),
        ],
        out_shape=[
            jax.ShapeDtypeStruct((N, HP), jnp.int32),
            jax.ShapeDtypeStruct((N, GW), jnp.float32),
            jax.ShapeDtypeStruct((N, GW), jnp.float32),
        ],
    )(x_src, x_dst, Wsrc, Wdst, asrc.reshape(1, HC), adst.reshape(1, HC))


@functools.partial(
    pl.kernel,
    mesh=_mesh,
    out_type=(
        jax.ShapeDtypeStruct((EPR, GW), jnp.float32),
        jax.ShapeDtypeStruct((NC, NP, GW), jnp.float32),
    ),
    scratch_types=[
        pltpu.VMEM((ICA, KEA), jnp.int32),
        pltpu.VMEM((ICA, KEA), jnp.int32),
        pltpu.VMEM((2, KEA, GW), jnp.float32),
        pltpu.VMEM((2, KEA, GW), jnp.float32),
        pltpu.VMEM((ICA * KEA // 8, GW), jnp.float32),
        pltpu.VMEM((KEA, GW), jnp.float32),
        pltpu.VMEM_SHARED((NP, GW), jnp.float32),
        pltpu.SemaphoreType.DMA((2,)),
        pltpu.SemaphoreType.DMA((2,)),
    ],
)
def _phase_a(src_hbm, dst_hbm, as_hbm, ad_hbm, ex_hbm, spart_hbm,
             idx_s, idx_d, asr, adr, exb2, exw, s_acc, sem1, sem2):
    cid = lax.axis_index("c")
    sid = lax.axis_index("s")
    wid = sid * NC + cid

    def zbody(k, _):
        for j in range(GW // AW):
            exw[k, pl.ds(j * AW, AW)] = jnp.zeros((AW,), jnp.float32)
        return 0
    lax.fori_loop(0, KEA, zbody, 0)

    def zcopy(t, _):
        pltpu.sync_copy(exw, s_acc.at[pl.ds(sid * RPT + t * KEA, KEA)])
        return 0
    lax.fori_loop(0, RPT // KEA, zcopy, 0)
    plsc.subcore_barrier()

    def issue(ii, b):
        pltpu.async_copy(as_hbm.at[idx_s.at[ii]], asr.at[b], sem1.at[b])
        pltpu.async_copy(ad_hbm.at[idx_d.at[ii]], adr.at[b], sem2.at[b])

    def wait(ii, b):
        pltpu.make_async_copy(as_hbm.at[idx_s.at[ii]], asr.at[b],
                              sem1.at[b]).wait()
        pltpu.make_async_copy(ad_hbm.at[idx_d.at[ii]], adr.at[b],
                              sem2.at[b]).wait()

    def outer(c, _):
        pltpu.sync_copy(src_hbm.at[wid, pl.ds(c * ICA, ICA)], idx_s)
        pltpu.sync_copy(dst_hbm.at[wid, pl.ds(c * ICA, ICA)], idx_d)
        issue(0, 0)

        def inner(j, _):
            for b in range(2):
                ii = 2 * j + b
                wait(ii, b)

                @pl.when(ii < ICA - 1)
                def _():
                    issue(ii + 1, 1 - b)

                for k in range(KEA):
                    v = (asr.at[b][k, pl.ds(0, AW)]
                         + adr.at[b][k, pl.ds(0, AW)])
                    v = jnp.maximum(v, 0.2 * v)
                    v = jnp.exp(v)
                    exw[k, pl.ds(0, AW)] = v
                    exb2[ii * (KEA // 8) + k // 8,
                         pl.ds((k % 8) * AW, AW)] = v
                pltpu.sync_copy(exw, s_acc.at[idx_d.at[ii]], add=True)
            return 0
        lax.fori_loop(0, ICA // 2, inner, 0)
        pltpu.sync_copy(exb2,
                        ex_hbm.at[pl.ds(wid * EPWR + c * (ICA * KEA // 8),
                                        ICA * KEA // 8)])
        return 0
    lax.fori_loop(0, NITA // ICA, outer, 0)

    plsc.subcore_barrier()

    def wcopy(t, _):
        r0 = sid * RPT + t * KEA
        pltpu.sync_copy(s_acc.at[pl.ds(r0, KEA)], exw)
        pltpu.sync_copy(exw, spart_hbm.at[cid, pl.ds(r0, KEA)])
        return 0
    lax.fori_loop(0, RPT // KEA, wcopy, 0)


def _ssum_body(p_ref, o_ref):
    o_ref[...] = p_ref[0] + p_ref[1]


def _ssum(parts):
    nb = 1024
    return pl.pallas_call(
        _ssum_body,
        grid=(NP // nb,),
        in_specs=[pl.BlockSpec((NC, nb, GW), lambda i: (0, i, 0))],
        out_specs=pl.BlockSpec((nb, GW), lambda i: (i, 0)),
        out_shape=jax.ShapeDtypeStruct((NP, GW), jnp.float32),
    )(parts)


@functools.partial(
    pl.kernel,
    mesh=_mesh,
    out_type=jax.ShapeDtypeStruct((EPR, GW), jnp.float32),
    scratch_types=[
        pltpu.VMEM((NIT2, KE2), jnp.int32),
        pltpu.VMEM((2, KE2, GW), jnp.float32),
        pltpu.VMEM((KE2 // 8, GW), jnp.float32),
        pltpu.VMEM((KE2 // 8, GW), jnp.float32),
        pltpu.SemaphoreType.DMA((2,)),
    ],
)
def _phase_a2(dst_hbm, ex_hbm, s_hbm, al_hbm,
              idx_d, ssr, exb, alb, sem):
    cid = lax.axis_index("c")
    sid = lax.axis_index("s")
    wid = sid * NC + cid

    pltpu.sync_copy(dst_hbm.at[wid], idx_d)
    pltpu.async_copy(s_hbm.at[idx_d.at[0]], ssr.at[0], sem.at[0])

    def body(j, _):
        for b in range(2):
            i = 2 * j + b

            @pl.when(i < NIT2 - 1)
            def _():
                pltpu.async_copy(s_hbm.at[idx_d.at[i + 1]], ssr.at[1 - b],
                                 sem.at[1 - b])
            pltpu.make_async_copy(s_hbm.at[idx_d.at[i]], ssr.at[b],
                                  sem.at[b]).wait()
            base = wid * EPWR + i * (KE2 // 8)
            pltpu.sync_copy(ex_hbm.at[pl.ds(base, KE2 // 8)], exb)
            for k in range(KE2):
                sv = ssr.at[b][k, pl.ds(0, AW)]
                ev = exb[k // 8, pl.ds((k % 8) * AW, AW)]
                alb[k // 8, pl.ds((k % 8) * AW, AW)] = (
                    ev * (1.0 / H) / (sv + 1e-16))
            pltpu.sync_copy(alb, al_hbm.at[pl.ds(base, KE2 // 8)])
        return 0
    lax.fori_loop(0, NIT2 // 2, body, 0)


@functools.partial(
    pl.kernel,
    mesh=_mesh,
    out_type=jax.ShapeDtypeStruct((NC, NP, C), jnp.float32),
    scratch_types=[
        pltpu.VMEM((ICB, KEB), jnp.int32),
        pltpu.VMEM((ICB, KEB), jnp.int32),
        pltpu.VMEM((ICB * KEB // 8, GW), jnp.float32),
        pltpu.VMEM((2, KEB, HP), jnp.int32),
        pltpu.VMEM((KEB, C), jnp.float32),
        pltpu.VMEM_SHARED((NP, C), jnp.float32),
        pltpu.SemaphoreType.DMA((2,)),
    ],
)
def _phase_b(src_hbm, dst_hbm, al_hbm, hs_hbm, opart_hbm,
             idx_s, idx_d, alb, hsb, msgb, out_acc, sem1):
    cid = lax.axis_index("c")
    sid = lax.axis_index("s")
    wid = sid * NC + cid

    def zbody(k, _):
        for j in range(C // AW):
            msgb[k, pl.ds(j * AW, AW)] = jnp.zeros((AW,), jnp.float32)
        return 0
    lax.fori_loop(0, KEB, zbody, 0)

    def zcopy(t, _):
        pltpu.sync_copy(msgb, out_acc.at[pl.ds(sid * RPT + t * KEB, KEB)])
        return 0
    lax.fori_loop(0, RPT // KEB, zcopy, 0)
    plsc.subcore_barrier()

    def issue(ii, b):
        pltpu.async_copy(hs_hbm.at[idx_s.at[ii]], hsb.at[b], sem1.at[b])

    def wait(ii, b):
        pltpu.make_async_copy(hs_hbm.at[idx_s.at[ii]], hsb.at[b],
                              sem1.at[b]).wait()

    def outer(c, _):
        pltpu.sync_copy(src_hbm.at[wid, pl.ds(c * ICB, ICB)], idx_s)
        pltpu.sync_copy(dst_hbm.at[wid, pl.ds(c * ICB, ICB)], idx_d)
        pltpu.sync_copy(
            al_hbm.at[pl.ds(wid * EPWR + c * (ICB * KEB // 8),
                            ICB * KEB // 8)], alb)
        issue(0, 0)

        def inner(j, _):
            for b in range(2):
                ii = 2 * j + b
                wait(ii, b)

                @pl.when(ii < ICB - 1)
                def _():
                    issue(ii + 1, 1 - b)

                for k in range(KEB):
                    av = alb[ii * (KEB // 8) + k // 8,
                             pl.ds((k % 8) * AW, AW)]
                    acca = [jnp.zeros((AW,), jnp.float32) for _ in range(4)]
                    accb = [jnp.zeros((AW,), jnp.float32) for _ in range(4)]
                    for h in range(H):
                        a = av[h]
                        for m in range(4):
                            vi = hsb.at[b][k, pl.ds(h * 64 + m * AW, AW)]
                            ua = lax.bitcast_convert_type(
                                vi << 16, jnp.float32)
                            ub = lax.bitcast_convert_type(
                                vi & jnp.int32(-65536), jnp.float32)
                            acca[m] = acca[m] + a * ua
                            accb[m] = accb[m] + a * ub
                    for m in range(4):
                        msgb[k, pl.ds(m * AW, AW)] = acca[m]
                        msgb[k, pl.ds(64 + m * AW, AW)] = accb[m]
                pltpu.sync_copy(msgb, out_acc.at[idx_d.at[ii]], add=True)
            return 0
        lax.fori_loop(0, ICB // 2, inner, 0)
        return 0
    lax.fori_loop(0, NITB // ICB, outer, 0)

    plsc.subcore_barrier()

    def wcopy(t, _):
        r0 = sid * RPT + t * KEB
        pltpu.sync_copy(out_acc.at[pl.ds(r0, KEB)], msgb)
        pltpu.sync_copy(msgb, opart_hbm.at[cid, pl.ds(r0, KEB)])
        return 0
    lax.fori_loop(0, RPT // KEB, wcopy, 0)


def _fin_body(p_ref, b_ref, w_ref, lb_ref, o_ref):
    t = p_ref[0] + p_ref[1] + b_ref[...]
    mu = jnp.mean(t, axis=-1, keepdims=True)
    var = jnp.mean((t - mu) * (t - mu), axis=-1, keepdims=True)
    y = (t - mu) / jnp.sqrt(var + 1e-5) * w_ref[...] + lb_ref[...]
    o_ref[...] = jnp.maximum(y, 0.0)


def _finalize(parts, b, lnw, lnb):
    nb = 1000
    return pl.pallas_call(
        _fin_body,
        grid=(N // nb,),
        in_specs=[
            pl.BlockSpec((NC, nb, C), lambda i: (0, i, 0)),
            pl.BlockSpec((1, C), lambda i: (0, 0)),
            pl.BlockSpec((1, C), lambda i: (0, 0)),
            pl.BlockSpec((1, C), lambda i: (0, 0)),
        ],
        out_specs=pl.BlockSpec((nb, C), lambda i: (i, 0)),
        out_shape=jax.ShapeDtypeStruct((N, C), jnp.float32),
    )(parts, b.reshape(1, C), lnw.reshape(1, C), lnb.reshape(1, C))


def _conv(x_src, x_dst, ei, Wsrc, Wdst, asrc, adst):
    sA, dA, d2, sB, dB = ei
    hs, a_s, a_d = _project(x_src, x_dst, Wsrc, Wdst, asrc, adst)
    ex, spart = _phase_a(sA, dA, a_s, a_d)
    s_sum = _ssum(spart)
    alpha = _phase_a2(d2, ex, s_sum)
    opart = _phase_b(sB, dB, alpha, hs)
    return opart


def _pad_edges(ei):
    npad = EP - E
    src = jnp.concatenate(
        [ei[0], (jnp.arange(npad, dtype=jnp.int32) * 37) % N])
    dst = jnp.concatenate(
        [ei[1], N + (jnp.arange(npad, dtype=jnp.int32) % (NP - N))])
    return (src.reshape(NW, NITA, KEA), dst.reshape(NW, NITA, KEA),
            dst.reshape(NW, NIT2, KE2),
            src.reshape(NW, NITB, KEB), dst.reshape(NW, NITB, KEB))


def kernel(x_user, x_item, edge_index_u2i, edge_index_i2u,
           Wsrc_0_u2i, Wdst_0_u2i, asrc_0_u2i, adst_0_u2i, b_0_u2i,
           Wsrc_0_i2u, Wdst_0_i2u, asrc_0_i2u, adst_0_i2u, b_0_i2u,
           lnw_0_user, lnb_0_user, lnw_0_item, lnb_0_item,
           Wsrc_1_u2i, Wdst_1_u2i, asrc_1_u2i, adst_1_u2i, b_1_u2i,
           Wsrc_1_i2u, Wdst_1_i2u, asrc_1_i2u, adst_1_i2u, b_1_i2u,
           lnw_1_user, lnb_1_user, lnw_1_item, lnb_1_item):
    p = dict(locals())
    ei_u2i = _pad_edges(edge_index_u2i)
    ei_i2u = _pad_edges(edge_index_i2u)
    xu, xi = x_user, x_item
    for l in range(2):
        op_i = _conv(xu, xi, ei_u2i,
                     p[f"Wsrc_{l}_u2i"], p[f"Wdst_{l}_u2i"],
                     p[f"asrc_{l}_u2i"], p[f"adst_{l}_u2i"])
        op_u = _conv(xi, xu, ei_i2u,
                     p[f"Wsrc_{l}_i2u"], p[f"Wdst_{l}_i2u"],
                     p[f"asrc_{l}_i2u"], p[f"adst_{l}_i2u"])
        xi = _finalize(op_i, p[f"b_{l}_u2i"], p[f"lnw_{l}_item"],
                       p[f"lnb_{l}_item"])
        xu = _finalize(op_u, p[f"b_{l}_i2u"], p[f"lnw_{l}_user"],
                       p[f"lnb_{l}_user"])
    return jnp.stack([xu, xi], axis=0)
